# Initial kernel scaffold; baseline (speedup 1.0000x reference)
import functools
import jax
import jax.numpy as jnp
from jax import lax
from jax.experimental import pallas as pl
from jax.experimental.pallas import tpu as pltpu
from jax.experimental.pallas import tpu_sc as plsc

B = 4096
L = 200
D = 64
LC = 100
NCHUNK = L // LC


def _sc_kernel(rid_hbm, e_hbm, table_hbm, out_hbm, idx_v, w_v, e_v, o_v, sem):
    info = plsc.get_sparse_core_info()
    nc = info.num_cores
    wid = lax.axis_index("s") * nc + lax.axis_index("c")
    b_per_w = B // (nc * info.num_subcores)
    base = wid * b_per_w

    def b_body(bi, _):
        b = base + bi
        for c in range(NCHUNK):
            pltpu.sync_copy(rid_hbm.at[b, pl.ds(c * LC, LC)], idx_v.at[c])
        cps = [
            pltpu.async_copy(table_hbm.at[idx_v.at[c]],
                             w_v.at[pl.ds(c * LC, LC)], sem)
            for c in range(NCHUNK)
        ]
        pltpu.sync_copy(e_hbm.at[b], e_v)
        for cp in cps:
            cp.wait()

        def l_body(l, carry):
            a0, a1, a2, a3 = carry
            w0 = w_v[l, pl.ds(0, 16)]
            w1 = w_v[l, pl.ds(16, 16)]
            w2 = w_v[l, pl.ds(32, 16)]
            w3 = w_v[l, pl.ds(48, 16)]
            e0 = e_v[l, pl.ds(0, 16)]
            e1 = e_v[l, pl.ds(16, 16)]
            e2 = e_v[l, pl.ds(32, 16)]
            e3 = e_v[l, pl.ds(48, 16)]
            s = jnp.sum(w0 * w0 + w1 * w1 + w2 * w2 + w3 * w3)
            d = jnp.sum(e0 * w0 + e1 * w1 + e2 * w2 + e3 * w3)
            coef = d / jnp.maximum(s, 1e-24)
            return (a0 + (e0 - coef * w0),
                    a1 + (e1 - coef * w1),
                    a2 + (e2 - coef * w2),
                    a3 + (e3 - coef * w3))

        z = jnp.zeros((16,), jnp.float32)
        a0, a1, a2, a3 = lax.fori_loop(0, L, l_body, (z, z, z, z))
        inv = jnp.float32(1.0 / L)
        o_v[pl.ds(0, 16)] = a0 * inv
        o_v[pl.ds(16, 16)] = a1 * inv
        o_v[pl.ds(32, 16)] = a2 * inv
        o_v[pl.ds(48, 16)] = a3 * inv
        pltpu.sync_copy(o_v, out_hbm.at[b])
        return 0

    lax.fori_loop(0, b_per_w, b_body, 0)


@jax.jit
def _run(batch_nei_rid, batch_nei_e_emb, w_r_table):
    mesh = plsc.VectorSubcoreMesh(core_axis_name="c", subcore_axis_name="s")
    kfn = functools.partial(
        pl.kernel,
        mesh=mesh,
        out_type=jax.ShapeDtypeStruct((B, D), jnp.float32),
        scratch_types=[
            pltpu.VMEM((NCHUNK, LC), jnp.int32),
            pltpu.VMEM((L, D), jnp.float32),
            pltpu.VMEM((L, D), jnp.float32),
            pltpu.VMEM((D,), jnp.float32),
            pltpu.SemaphoreType.DMA,
        ],
    )(_sc_kernel)
    return kfn(batch_nei_rid, batch_nei_e_emb, w_r_table)


def kernel(batch_nei_rid, batch_nei_e_emb, w_r_table):
    return _run(batch_nei_rid, batch_nei_e_emb, w_r_table)


# speedup vs baseline: 13.5662x; 13.5662x over previous
import functools
import jax
import jax.numpy as jnp
from jax import lax
from jax.experimental import pallas as pl
from jax.experimental.pallas import tpu as pltpu
from jax.experimental.pallas import tpu_sc as plsc

B = 4096
L = 200
D = 64
CHUNKS = (104, 96)


def _sc_kernel(rid_hbm, e_hbm, table_hbm, out_hbm, idx_v, w_v, e_v, o_v, sem):
    info = plsc.get_sparse_core_info()
    nc = info.num_cores
    wid = lax.axis_index("s") * nc + lax.axis_index("c")
    b_per_w = B // (nc * info.num_subcores)
    base = wid * b_per_w

    def b_body(bi, _):
        b = base + bi
        pltpu.sync_copy(rid_hbm.at[pl.ds(b * L, L)], idx_v)
        cps = []
        off = 0
        for c in CHUNKS:
            cps.append(pltpu.async_copy(table_hbm.at[idx_v.at[pl.ds(off, c)]],
                                        w_v.at[pl.ds(off, c)], sem))
            off += c
        pltpu.sync_copy(e_hbm.at[b], e_v)
        for cp in cps:
            cp.wait()

        rot = [(jnp.arange(16, dtype=jnp.int32) + sh) & 15 for sh in (8, 4, 2, 1)]

        dnums = lax.GatherDimensionNumbers(
            offset_dims=(), collapsed_slice_dims=(0,), start_index_map=(0,))

        def _allsum(v):
            for idx in rot:
                p = lax.gather(v, idx[:, None], dnums, (1,),
                               mode=lax.GatherScatterMode.PROMISE_IN_BOUNDS)
                v = v + p
            return v

        def l_body(l, carry):
            a0, a1, a2, a3 = carry
            w0 = w_v[l, pl.ds(0, 16)]
            w1 = w_v[l, pl.ds(16, 16)]
            w2 = w_v[l, pl.ds(32, 16)]
            w3 = w_v[l, pl.ds(48, 16)]
            e0 = e_v[l, pl.ds(0, 16)]
            e1 = e_v[l, pl.ds(16, 16)]
            e2 = e_v[l, pl.ds(32, 16)]
            e3 = e_v[l, pl.ds(48, 16)]
            s = _allsum(w0 * w0 + w1 * w1 + w2 * w2 + w3 * w3)
            d = _allsum(e0 * w0 + e1 * w1 + e2 * w2 + e3 * w3)
            coef = d / jnp.maximum(s, 1e-24)
            return (a0 + (e0 - coef * w0),
                    a1 + (e1 - coef * w1),
                    a2 + (e2 - coef * w2),
                    a3 + (e3 - coef * w3))

        z = jnp.zeros((16,), jnp.float32)
        a0, a1, a2, a3 = lax.fori_loop(0, L, l_body, (z, z, z, z))
        inv = jnp.float32(1.0 / L)
        o_v[pl.ds(0, 16)] = a0 * inv
        o_v[pl.ds(16, 16)] = a1 * inv
        o_v[pl.ds(32, 16)] = a2 * inv
        o_v[pl.ds(48, 16)] = a3 * inv
        pltpu.sync_copy(o_v, out_hbm.at[pl.ds(b * D, D)])
        return 0

    lax.fori_loop(0, b_per_w, b_body, 0)


@jax.jit
def _run(rid_flat, batch_nei_e_emb, w_r_table):
    mesh = plsc.VectorSubcoreMesh(core_axis_name="c", subcore_axis_name="s")
    kfn = functools.partial(
        pl.kernel,
        mesh=mesh,
        compiler_params=pltpu.CompilerParams(use_tc_tiling_on_sc=False),
        out_type=jax.ShapeDtypeStruct((B * D,), jnp.float32),
        scratch_types=[
            pltpu.VMEM((L,), jnp.int32),
            pltpu.VMEM((L, D), jnp.float32),
            pltpu.VMEM((L, D), jnp.float32),
            pltpu.VMEM((D,), jnp.float32),
            pltpu.SemaphoreType.DMA,
        ],
    )(_sc_kernel)
    return kfn(rid_flat, batch_nei_e_emb, w_r_table).reshape(B, D)


def kernel(batch_nei_rid, batch_nei_e_emb, w_r_table):
    return _run(batch_nei_rid.reshape(-1), batch_nei_e_emb, w_r_table)


# speedup vs baseline: 17.8913x; 1.3188x over previous
import functools
import jax
import jax.numpy as jnp
from jax import lax
from jax.experimental import pallas as pl
from jax.experimental.pallas import tpu as pltpu
from jax.experimental.pallas import tpu_sc as plsc

B = 4096
L = 200
D = 64
CHUNKS = ((0, 104), (104, 96))
NBUF = 2


def _sc_kernel(rid_hbm, e_hbm, table_hbm, out_hbm,
               idx_all, w_v, e_v, o_v, sems):
    info = plsc.get_sparse_core_info()
    nc = info.num_cores
    wid = lax.axis_index("s") * nc + lax.axis_index("c")
    b_per_w = B // (nc * info.num_subcores)
    base = wid * b_per_w

    pltpu.sync_copy(rid_hbm.at[pl.ds(base * L, b_per_w * L)], idx_all)

    def issue(bi, slot):
        for off, c in CHUNKS:
            pltpu.async_copy(
                table_hbm.at[idx_all.at[pl.ds(bi * L + off, c)]],
                w_v.at[slot].at[pl.ds(off, c)], sems.at[slot])
        pltpu.async_copy(e_hbm.at[base + bi], e_v.at[slot], sems.at[slot])

    def drain(slot):
        for off, c in CHUNKS:
            pltpu.make_async_copy(e_hbm.at[0].at[pl.ds(0, c)],
                                  w_v.at[slot].at[pl.ds(off, c)],
                                  sems.at[slot]).wait()
        pltpu.make_async_copy(e_hbm.at[0], e_v.at[slot],
                              sems.at[slot]).wait()

    rot = [(jnp.arange(16, dtype=jnp.int32) + sh) & 15 for sh in (8, 4, 2, 1)]
    dnums = lax.GatherDimensionNumbers(
        offset_dims=(), collapsed_slice_dims=(0,), start_index_map=(0,))

    def _allsum(v):
        for idx in rot:
            p = lax.gather(v, idx[:, None], dnums, (1,),
                           mode=lax.GatherScatterMode.PROMISE_IN_BOUNDS)
            v = v + p
        return v

    def compute(slot, b):
        wb = w_v.at[slot]
        eb = e_v.at[slot]

        def l_body(l2, carry):
            a0, a1, a2, a3 = carry
            for u in range(2):
                l = l2 * 2 + u
                w0 = wb[l, pl.ds(0, 16)]
                w1 = wb[l, pl.ds(16, 16)]
                w2 = wb[l, pl.ds(32, 16)]
                w3 = wb[l, pl.ds(48, 16)]
                e0 = eb[l, pl.ds(0, 16)]
                e1 = eb[l, pl.ds(16, 16)]
                e2 = eb[l, pl.ds(32, 16)]
                e3 = eb[l, pl.ds(48, 16)]
                s = _allsum(w0 * w0 + w1 * w1 + w2 * w2 + w3 * w3)
                d = _allsum(e0 * w0 + e1 * w1 + e2 * w2 + e3 * w3)
                coef = d / jnp.maximum(s, 1e-24)
                a0 = a0 + (e0 - coef * w0)
                a1 = a1 + (e1 - coef * w1)
                a2 = a2 + (e2 - coef * w2)
                a3 = a3 + (e3 - coef * w3)
            return (a0, a1, a2, a3)

        z = jnp.zeros((16,), jnp.float32)
        a0, a1, a2, a3 = lax.fori_loop(0, L // 2, l_body, (z, z, z, z))
        inv = jnp.float32(1.0 / L)
        o_v[pl.ds(0, 16)] = a0 * inv
        o_v[pl.ds(16, 16)] = a1 * inv
        o_v[pl.ds(32, 16)] = a2 * inv
        o_v[pl.ds(48, 16)] = a3 * inv
        pltpu.sync_copy(o_v, out_hbm.at[pl.ds(b * D, D)])

    issue(0, 0)

    def pair_body(p, _):
        for s2 in range(NBUF):
            bi = p * NBUF + s2
            drain(s2)

            @pl.when(bi + 1 < b_per_w)
            def _():
                issue(bi + 1, (s2 + 1) % NBUF)

            compute(s2, base + bi)
        return 0

    lax.fori_loop(0, b_per_w // NBUF, pair_body, 0)


@jax.jit
def _run(rid_flat, batch_nei_e_emb, w_r_table):
    mesh = plsc.VectorSubcoreMesh(core_axis_name="c", subcore_axis_name="s")
    kfn = functools.partial(
        pl.kernel,
        mesh=mesh,
        compiler_params=pltpu.CompilerParams(use_tc_tiling_on_sc=False),
        out_type=jax.ShapeDtypeStruct((B * D,), jnp.float32),
        scratch_types=[
            pltpu.VMEM((B // 32 * L,), jnp.int32),
            pltpu.VMEM((NBUF, L, D), jnp.float32),
            pltpu.VMEM((NBUF, L, D), jnp.float32),
            pltpu.VMEM((D,), jnp.float32),
            pltpu.SemaphoreType.DMA((NBUF,)),
        ],
    )(_sc_kernel)
    return kfn(rid_flat, batch_nei_e_emb, w_r_table).reshape(B, D)


def kernel(batch_nei_rid, batch_nei_e_emb, w_r_table):
    return _run(batch_nei_rid.reshape(-1), batch_nei_e_emb, w_r_table)
